# baseline (speedup 1.0000x reference)
import functools

import jax
import jax.numpy as jnp
from jax import lax
from jax.experimental import pallas as pl
from jax.experimental.pallas import tpu as pltpu
from jax.experimental.pallas import tpu_sc as plsc

D = 64
NC = 2
NS = 16
NW = NC * NS
CH = 128
L = 16


def _rsqrt(x):
    i = plsc.bitcast(x, jnp.int32)
    i = jnp.int32(0x5F3759DF) - (i >> 1)
    y = plsc.bitcast(i, jnp.float32)
    hx = x * jnp.float32(0.5)
    for _ in range(3):
        y = y * (jnp.float32(1.5) - hx * y * y)
    return y


def _tile_body(nch, head_hbm, rel_hbm, tail_hbm, ent_hbm, relemb_hbm,
               out_hbm, hidx, ridx, tidx, hbuf, rbuf, tbuf, sbuf, sem):
    wid = lax.axis_index("s") * NC + lax.axis_index("c")
    base = wid * (nch * CH)

    pltpu.sync_copy(head_hbm.at[wid], hidx)
    pltpu.sync_copy(rel_hbm.at[wid], ridx)
    pltpu.sync_copy(tail_hbm.at[wid], tidx)

    def gather(c, buf_slot):
        return (
            pltpu.async_copy(ent_hbm.at[hidx.at[c]], hbuf.at[buf_slot], sem),
            pltpu.async_copy(relemb_hbm.at[ridx.at[c]], rbuf.at[buf_slot], sem),
            pltpu.async_copy(ent_hbm.at[tidx.at[c]], tbuf.at[buf_slot], sem),
        )

    cps = gather(0, 0)
    for c in range(nch):
        slot = c % 2
        for cp in cps:
            cp.wait()
        if c + 1 < nch:
            cps = gather(c + 1, 1 - slot)

        for g in range(CH // L):
            row = lax.iota(jnp.int32, L) + g * L
            zeros = jnp.zeros((L,), jnp.float32)

            def step(j, acc):
                sh, sr, st, xhr, xht, xrt = acc
                col = jnp.full((L,), j, jnp.int32)
                gh = plsc.load_gather(hbuf.at[slot], [row, col])
                gr = plsc.load_gather(rbuf.at[slot], [row, col])
                gt = plsc.load_gather(tbuf.at[slot], [row, col])
                return (sh + gh * gh, sr + gr * gr, st + gt * gt,
                        xhr + gh * gr, xht + gh * gt, xrt + gr * gt)

            sh, sr, st, xhr, xht, xrt = lax.fori_loop(
                0, D, step, (zeros,) * 6)

            cap = jnp.full((L,), 1e12, jnp.float32)
            ih = jnp.minimum(_rsqrt(sh), cap)
            ir = jnp.minimum(_rsqrt(sr), cap)
            it = jnp.minimum(_rsqrt(st), cap)
            s2 = (sh * ih * ih + sr * ir * ir + st * it * it
                  + jnp.float32(2.0) * (xhr * ih * ir - xht * ih * it
                                        - xrt * ir * it))
            s2 = jnp.maximum(s2, jnp.float32(0.0))
            sbuf[pl.ds(c * CH + g * L, L)] = s2 * _rsqrt(s2)

    pltpu.sync_copy(sbuf, out_hbm.at[pl.ds(base, nch * CH)])


def kernel(head, relation, tail, entity_embeddings, relation_embeddings):
    b = head.shape[0]
    nch = b // (NW * CH)
    mesh = plsc.VectorSubcoreMesh(core_axis_name="c", subcore_axis_name="s")
    f = functools.partial(
        pl.kernel,
        out_type=jax.ShapeDtypeStruct((b,), jnp.float32),
        mesh=mesh,
        compiler_params=pltpu.CompilerParams(needs_layout_passes=False,
                                             use_tc_tiling_on_sc=False),
        scratch_types=[
            pltpu.VMEM((nch, CH), jnp.int32),
            pltpu.VMEM((nch, CH), jnp.int32),
            pltpu.VMEM((nch, CH), jnp.int32),
            pltpu.VMEM((2, CH, D), jnp.float32),
            pltpu.VMEM((2, CH, D), jnp.float32),
            pltpu.VMEM((2, CH, D), jnp.float32),
            pltpu.VMEM((nch * CH,), jnp.float32),
            pltpu.SemaphoreType.DMA,
        ],
    )(functools.partial(_tile_body, nch))
    h3 = head.reshape(NW, nch, CH)
    r3 = relation.reshape(NW, nch, CH)
    t3 = tail.reshape(NW, nch, CH)
    return f(h3, r3, t3, entity_embeddings, relation_embeddings)


# speedup vs baseline: 1.0000x; 1.0000x over previous
import functools

import jax
import jax.numpy as jnp
from jax import lax
from jax.experimental import pallas as pl
from jax.experimental.pallas import tpu as pltpu
from jax.experimental.pallas import tpu_sc as plsc

D = 64
NC = 2
NS = 16
NW = NC * NS
CH = 128
L = 16
UNROLL = 4


def _rsqrt(x):
    i = plsc.bitcast(x, jnp.int32)
    i = jnp.int32(0x5F3759DF) - (i >> 1)
    y = plsc.bitcast(i, jnp.float32)
    hx = x * jnp.float32(0.5)
    for _ in range(3):
        y = y * (jnp.float32(1.5) - hx * y * y)
    return y


def _tile_body(nch, hrow_hbm, hpar_hbm, rrow_hbm, rpar_hbm, trow_hbm,
               tpar_hbm, ent_hbm, rel_hbm, out_hbm,
               hidx, hpar, ridx, rpar, tidx, tpar, hbuf, rbuf, tbuf, sbuf,
               sem):
    wid = lax.axis_index("s") * NC + lax.axis_index("c")
    base = wid * (nch * CH)

    pltpu.sync_copy(hrow_hbm.at[wid], hidx)
    pltpu.sync_copy(hpar_hbm.at[wid], hpar)
    pltpu.sync_copy(rrow_hbm.at[wid], ridx)
    pltpu.sync_copy(rpar_hbm.at[wid], rpar)
    pltpu.sync_copy(trow_hbm.at[wid], tidx)
    pltpu.sync_copy(tpar_hbm.at[wid], tpar)

    def gather(c, slot):
        return (
            pltpu.async_copy(ent_hbm.at[hidx.at[c]], hbuf.at[slot], sem),
            pltpu.async_copy(rel_hbm.at[ridx.at[c]], rbuf.at[slot], sem),
            pltpu.async_copy(ent_hbm.at[tidx.at[c]], tbuf.at[slot], sem),
        )

    cps = gather(0, 0)
    for c in range(nch):
        slot = c % 2
        for cp in cps:
            cp.wait()
        if c + 1 < nch:
            cps = gather(c + 1, 1 - slot)

        for g in range(CH // L):
            row = lax.iota(jnp.int32, L) + g * L
            ph = hpar[c, pl.ds(g * L, L)] << 6
            pr = rpar[c, pl.ds(g * L, L)] << 6
            pt = tpar[c, pl.ds(g * L, L)] << 6
            zeros = jnp.zeros((L,), jnp.float32)

            def step(i, acc):
                sh, sr, st, xhr, xht, xrt = acc
                for u in range(UNROLL):
                    j = i * UNROLL + u
                    ch_ = ph + j
                    cr_ = pr + j
                    ct_ = pt + j
                    gh = plsc.load_gather(hbuf.at[slot], [row, ch_])
                    gr = plsc.load_gather(rbuf.at[slot], [row, cr_])
                    gt = plsc.load_gather(tbuf.at[slot], [row, ct_])
                    sh = sh + gh * gh
                    sr = sr + gr * gr
                    st = st + gt * gt
                    xhr = xhr + gh * gr
                    xht = xht + gh * gt
                    xrt = xrt + gr * gt
                return (sh, sr, st, xhr, xht, xrt)

            sh, sr, st, xhr, xht, xrt = lax.fori_loop(
                0, D // UNROLL, step, (zeros,) * 6)

            cap = jnp.full((L,), 1e12, jnp.float32)
            ih = jnp.minimum(_rsqrt(sh), cap)
            ir = jnp.minimum(_rsqrt(sr), cap)
            it = jnp.minimum(_rsqrt(st), cap)
            s2 = (sh * ih * ih + sr * ir * ir + st * it * it
                  + jnp.float32(2.0) * (xhr * ih * ir - xht * ih * it
                                        - xrt * ir * it))
            s2 = jnp.maximum(s2, jnp.float32(0.0))
            sbuf[pl.ds(c * CH + g * L, L)] = s2 * _rsqrt(s2)

    pltpu.sync_copy(sbuf, out_hbm.at[pl.ds(base, nch * CH)])


def kernel(head, relation, tail, entity_embeddings, relation_embeddings):
    b = head.shape[0]
    nch = b // (NW * CH)
    ne, d = entity_embeddings.shape
    nr = relation_embeddings.shape[0]
    ent2 = entity_embeddings.reshape(ne // 2, 2 * d)
    rel2 = relation_embeddings.reshape(nr // 2, 2 * d)

    def split(idx):
        return ((idx >> 1).reshape(NW, nch, CH),
                (idx & 1).reshape(NW, nch, CH))

    hrow, hpar = split(head)
    rrow, rpar = split(relation)
    trow, tpar = split(tail)

    mesh = plsc.VectorSubcoreMesh(core_axis_name="c", subcore_axis_name="s")
    f = functools.partial(
        pl.kernel,
        out_type=jax.ShapeDtypeStruct((b,), jnp.float32),
        mesh=mesh,
        compiler_params=pltpu.CompilerParams(needs_layout_passes=False,
                                             use_tc_tiling_on_sc=True),
        scratch_types=[
            pltpu.VMEM((nch, CH), jnp.int32),
            pltpu.VMEM((nch, CH), jnp.int32),
            pltpu.VMEM((nch, CH), jnp.int32),
            pltpu.VMEM((nch, CH), jnp.int32),
            pltpu.VMEM((nch, CH), jnp.int32),
            pltpu.VMEM((nch, CH), jnp.int32),
            pltpu.VMEM((2, CH, 2 * D), jnp.float32),
            pltpu.VMEM((2, CH, 2 * D), jnp.float32),
            pltpu.VMEM((2, CH, 2 * D), jnp.float32),
            pltpu.VMEM((nch * CH,), jnp.float32),
            pltpu.SemaphoreType.DMA,
        ],
    )(functools.partial(_tile_body, nch))
    return f(hrow, hpar, rrow, rpar, trow, tpar, ent2, rel2)


# speedup vs baseline: 1.1932x; 1.1932x over previous
import functools

import jax
import jax.numpy as jnp
from jax import lax
from jax.experimental import pallas as pl
from jax.experimental.pallas import tpu as pltpu
from jax.experimental.pallas import tpu_sc as plsc

D = 64
NC = 2
NS = 16
NW = NC * NS
CH = 128
L = 16
BLK = 128
SEG = 2664
MAXG = 2112
UNROLL = 4

_CP = pltpu.CompilerParams(needs_layout_passes=False, use_tc_tiling_on_sc=True)
_MESH = dict(core_axis_name="c", subcore_axis_name="s")


def _rsqrt(x):
    i = plsc.bitcast(x, jnp.int32)
    i = jnp.int32(0x5F3759DF) - (i >> 1)
    y = plsc.bitcast(i, jnp.float32)
    hx = x * jnp.float32(0.5)
    for _ in range(3):
        y = y * (jnp.float32(1.5) - hx * y * y)
    return y


def _sweep_body(sent_hbm, meta_hbm, cnts_hbm, entT_hbm, mat_hbm,
                sent_v, meta_v, cnt_v, bbuf, stage, semb, semf):
    wid = lax.axis_index("s") * NC + lax.axis_index("c")
    pltpu.sync_copy(meta_hbm.at[wid], meta_v)
    mv = meta_v[pl.ds(0, L)]
    b0 = mv[0]
    nblk = mv[1]
    salign = mv[3]
    doff = mv[4]
    g0 = mv[5]

    pltpu.sync_copy(cnts_hbm.at[wid], cnt_v)
    pltpu.sync_copy(
        sent_hbm.at[pl.ds(pl.multiple_of(salign, 8), SEG)], sent_v)

    iotas = [lax.iota(jnp.int32, L) + u * L for u in range(4)]

    def start_block(k):
        off = pl.multiple_of((b0 + k) * BLK, BLK)
        return pltpu.async_copy(
            entT_hbm.at[:, pl.ds(off, BLK)], bbuf.at[k & 1], semb)

    start_block(0)

    def block_body(k, carry):
        p, f, fl = carry
        pltpu.make_async_copy(
            entT_hbm.at[:, pl.ds(0, BLK)], bbuf.at[0], semb).wait()

        @pl.when(k + 1 < nblk)
        def _():
            start_block(k + 1)

        cnt = cnt_v[pl.ds(k, L)][0]
        slot = jnp.full((L,), k & 1, jnp.int32)

        def req_body(q, c2):
            f, fl = c2
            e = sent_v[pl.ds(doff + p + q, L)][0]
            col = jnp.full((L,), e & (BLK - 1), jnp.int32)
            srow = jnp.full((L,), f, jnp.int32)
            ss = jnp.full((L,), fl & 1, jnp.int32)
            for u in range(4):
                g = plsc.load_gather(bbuf, [slot, iotas[u], col])
                plsc.store_scatter(stage, [ss, srow, iotas[0] + u * L], g)
            f = f + 1

            def flush(c3):
                f, fl = c3

                @pl.when(fl >= 1)
                def _():
                    pltpu.make_async_copy(
                        stage.at[0], mat_hbm.at[0], semf).wait()

                pltpu.async_copy(
                    stage.at[fl & 1], mat_hbm.at[g0 + fl], semf)
                return (jnp.int32(0), fl + 1)

            return lax.cond(f >= L, flush, lambda c3: c3, (f, fl))

        f, fl = lax.fori_loop(0, cnt, req_body, (f, fl))
        return (p + cnt, f, fl)

    p, f, fl = lax.fori_loop(0, nblk, block_body,
                             (jnp.int32(0), jnp.int32(0), jnp.int32(0)))

    @pl.when(f > 0)
    def _():
        @pl.when(fl >= 1)
        def _():
            pltpu.make_async_copy(stage.at[0], mat_hbm.at[0], semf).wait()
        pltpu.async_copy(stage.at[fl & 1], mat_hbm.at[g0 + fl], semf).wait()

    @pl.when((f == 0) & (fl >= 1))
    def _():
        pltpu.make_async_copy(stage.at[0], mat_hbm.at[0], semf).wait()


def _score_body(nch, hpos_hbm, tpos_hbm, rrow_hbm, rpar_hbm, mat_hbm,
                rel_hbm, out_hbm, hidx, tidx, ridx, rpar, hbuf, tbuf, rbuf,
                sbuf, sem):
    wid = lax.axis_index("s") * NC + lax.axis_index("c")
    base = wid * (nch * CH)

    pltpu.sync_copy(hpos_hbm.at[wid], hidx)
    pltpu.sync_copy(tpos_hbm.at[wid], tidx)
    pltpu.sync_copy(rrow_hbm.at[wid], ridx)
    pltpu.sync_copy(rpar_hbm.at[wid], rpar)

    def gather(c, slot):
        return (
            pltpu.async_copy(mat_hbm.at[hidx.at[c]], hbuf.at[slot], sem),
            pltpu.async_copy(mat_hbm.at[tidx.at[c]], tbuf.at[slot], sem),
            pltpu.async_copy(rel_hbm.at[ridx.at[c]], rbuf.at[slot], sem),
        )

    cps = gather(0, 0)
    for c in range(nch):
        slot = c % 2
        for cp in cps:
            cp.wait()
        if c + 1 < nch:
            cps = gather(c + 1, 1 - slot)

        for g in range(CH // L):
            row = lax.iota(jnp.int32, L) + g * L
            pr = rpar[c, pl.ds(g * L, L)] << 6
            zeros = jnp.zeros((L,), jnp.float32)

            def step(i, acc):
                sh, sr, st, xhr, xht, xrt = acc
                for u in range(UNROLL):
                    j = i * UNROLL + u
                    gh = plsc.load_gather(hbuf.at[slot], [row, jnp.full((L,), j, jnp.int32)])
                    gt = plsc.load_gather(tbuf.at[slot], [row, jnp.full((L,), j, jnp.int32)])
                    gr = plsc.load_gather(rbuf.at[slot], [row, pr + j])
                    sh = sh + gh * gh
                    sr = sr + gr * gr
                    st = st + gt * gt
                    xhr = xhr + gh * gr
                    xht = xht + gh * gt
                    xrt = xrt + gr * gt
                return (sh, sr, st, xhr, xht, xrt)

            sh, sr, st, xhr, xht, xrt = lax.fori_loop(
                0, D // UNROLL, step, (zeros,) * 6)

            cap = jnp.full((L,), 1e12, jnp.float32)
            ih = jnp.minimum(_rsqrt(sh), cap)
            ir = jnp.minimum(_rsqrt(sr), cap)
            it = jnp.minimum(_rsqrt(st), cap)
            s2 = (sh * ih * ih + sr * ir * ir + st * it * it
                  + jnp.float32(2.0) * (xhr * ih * ir - xht * ih * it
                                        - xrt * ir * it))
            s2 = jnp.maximum(s2, jnp.float32(0.0))
            sbuf[pl.ds(c * CH + g * L, L)] = s2 * _rsqrt(s2)

    pltpu.sync_copy(sbuf, out_hbm.at[pl.ds(base, nch * CH)])


def kernel(head, relation, tail, entity_embeddings, relation_embeddings):
    b = head.shape[0]
    nch = b // (NW * CH)
    ne = entity_embeddings.shape[0]
    nblk_total = (ne + BLK - 1) // BLK

    ent_all = jnp.concatenate([head, tail])
    order = jnp.argsort(ent_all)
    sorted_ent = ent_all[order]

    per = nblk_total // NW
    extra = nblk_total - per * NW
    w = jnp.arange(NW + 1, dtype=jnp.int32)
    bstart = w * per + jnp.minimum(w, extra)
    s = jnp.searchsorted(sorted_ent, (bstart * BLK).astype(sorted_ent.dtype),
                         side="left").astype(jnp.int32)
    n = s[1:] - s[:-1]
    mg = (n + (L - 1)) // L + 1
    g0 = jnp.concatenate([jnp.zeros((1,), jnp.int32),
                          jnp.cumsum(mg)])[:NW].astype(jnp.int32)
    salign = (s[:-1] // 8) * 8
    doff = s[:-1] - salign

    meta = jnp.stack([bstart[:-1], bstart[1:] - bstart[:-1], s[:-1],
                      salign, doff, g0] +
                     [jnp.zeros((NW,), jnp.int32)] * 10, axis=1)

    blk_of = (sorted_ent >> 7).astype(jnp.int32)
    counts = jnp.zeros((nblk_total,), jnp.int32).at[blk_of].add(1)
    cidx = jnp.minimum(bstart[:NW, None] + jnp.arange(272)[None, :],
                       nblk_total - 1)
    valid = jnp.arange(272)[None, :] < (bstart[1:, None] - bstart[:NW, None])
    cnts = jnp.where(valid, counts[cidx], 0).astype(jnp.int32)

    j = jnp.arange(2 * b, dtype=jnp.int32)
    wj = jnp.searchsorted(s[1:], j, side="right").astype(jnp.int32)
    pos_sorted = g0[wj] * L + (j - s[wj])
    matrow = jnp.zeros((2 * b,), jnp.int32).at[order].set(pos_sorted)
    hpos = matrow[:b].reshape(NW, nch, CH)
    tpos = matrow[b:].reshape(NW, nch, CH)

    sent_pad = jnp.concatenate(
        [sorted_ent.astype(jnp.int32),
         jnp.zeros((SEG + 8,), jnp.int32)])

    entT = entity_embeddings.T

    mesh_a = plsc.VectorSubcoreMesh(**_MESH)
    sweep = functools.partial(
        pl.kernel,
        out_type=jax.ShapeDtypeStruct((MAXG, L, BLK), jnp.float32),
        mesh=mesh_a,
        compiler_params=_CP,
        scratch_types=[
            pltpu.VMEM((SEG,), jnp.int32),
            pltpu.VMEM((L,), jnp.int32),
            pltpu.VMEM((272,), jnp.int32),
            pltpu.VMEM((2, D, BLK), jnp.float32),
            pltpu.VMEM((2, L, BLK), jnp.float32),
            pltpu.SemaphoreType.DMA,
            pltpu.SemaphoreType.DMA,
        ],
    )(_sweep_body)
    mat = sweep(sent_pad, meta, cnts, entT)
    mat2 = mat.reshape(MAXG * L, BLK)

    nr = relation_embeddings.shape[0]
    rel2 = relation_embeddings.reshape(nr // 2, 2 * D)
    rrow = (relation >> 1).reshape(NW, nch, CH)
    rpar = (relation & 1).reshape(NW, nch, CH)

    mesh_b = plsc.VectorSubcoreMesh(**_MESH)
    score = functools.partial(
        pl.kernel,
        out_type=jax.ShapeDtypeStruct((b,), jnp.float32),
        mesh=mesh_b,
        compiler_params=_CP,
        scratch_types=[
            pltpu.VMEM((nch, CH), jnp.int32),
            pltpu.VMEM((nch, CH), jnp.int32),
            pltpu.VMEM((nch, CH), jnp.int32),
            pltpu.VMEM((nch, CH), jnp.int32),
            pltpu.VMEM((2, CH, BLK), jnp.float32),
            pltpu.VMEM((2, CH, BLK), jnp.float32),
            pltpu.VMEM((2, CH, 2 * D), jnp.float32),
            pltpu.VMEM((nch * CH,), jnp.float32),
            pltpu.SemaphoreType.DMA,
        ],
    )(functools.partial(_score_body, nch))
    return score(hpos, tpos, rrow, rpar, mat2, rel2)


# speedup vs baseline: 1.9296x; 1.6171x over previous
import functools

import jax
import jax.numpy as jnp
from jax import lax
from jax.experimental import pallas as pl
from jax.experimental.pallas import tpu as pltpu
from jax.experimental.pallas import tpu_sc as plsc

D = 64
NC = 2
NS = 16
NW = NC * NS
CH = 128
L = 16
CW = 512
NE = 1000000
NFULL = NE // CW
TAIL0 = NFULL * CW
SEG = 2664
MAXG = 2112
UNROLL = 4

_CP = pltpu.CompilerParams(needs_layout_passes=False, use_tc_tiling_on_sc=True)
_MESH = dict(core_axis_name="c", subcore_axis_name="s")


def _rsqrt(x):
    i = plsc.bitcast(x, jnp.int32)
    i = jnp.int32(0x5F3759DF) - (i >> 1)
    y = plsc.bitcast(i, jnp.float32)
    hx = x * jnp.float32(0.5)
    for _ in range(3):
        y = y * (jnp.float32(1.5) - hx * y * y)
    return y


def _sweep_body(sent_hbm, meta_hbm, cnts_hbm, entT_hbm, tailT_hbm, mat_hbm,
                sent_v, meta_v, cnt_v, bbuf, tbuf, stage, semb, semf):
    wid = lax.axis_index("s") * NC + lax.axis_index("c")
    pltpu.sync_copy(meta_hbm.at[wid], meta_v)
    mv = meta_v[pl.ds(0, L)]
    c0 = mv[0]
    nck = mv[1]
    doff = mv[4]
    salign = mv[3]
    g0 = mv[5]

    pltpu.sync_copy(cnts_hbm.at[wid], cnt_v)
    pltpu.sync_copy(
        sent_hbm.at[pl.ds(pl.multiple_of(salign, 8), SEG)], sent_v)

    iota = lax.iota(jnp.int32, L)

    def flush_group(g, fg):
        @pl.when(fg >= 2)
        def _():
            pltpu.make_async_copy(stage.at[0], mat_hbm.at[0], semf).wait()
        pltpu.async_copy(stage.at[g & 3], mat_hbm.at[g0 + g], semf)

    def passes(buf, slot_s, start, wmax, cnt, p, fg):
        slotv = jnp.full((L,), slot_s, jnp.int32)

        def pass_body(ps, fg_):
            t0 = p + ps * L
            rem = jnp.minimum(cnt - ps * L, L)
            ev = sent_v[pl.ds(doff + t0, L)]
            cv = jnp.clip(ev - start, 0, wmax)
            tv = t0 + iota
            ssv = (tv >> 4) & 3
            rowv = tv & (L - 1)

            def jstep(i, _):
                for u in range(UNROLL):
                    jv = (jnp.full((L,), i * UNROLL + u, jnp.int32)
                          + iota) & (D - 1)
                    g = plsc.load_gather(buf, [slotv, jv, cv])
                    plsc.store_scatter(stage, [ssv, rowv, jv], g)
                return 0

            lax.fori_loop(0, D // UNROLL, jstep, 0)

            def doflush(fg2):
                flush_group(fg2, fg2)
                return fg2 + 1

            return lax.cond(((t0 + rem) >> 4) > fg_, doflush,
                            lambda fg2: fg2, fg_)

        fgo = lax.fori_loop(0, (cnt + L - 1) // L, pass_body, fg)
        return (p + cnt, fgo)

    def start_chunk(k):
        off = pl.multiple_of((c0 + k) * CW, CW)
        return pltpu.async_copy(
            entT_hbm.at[:, pl.ds(off, CW)], bbuf.at[k & 1], semb)

    @pl.when(nck > 0)
    def _():
        start_chunk(0)

    def chunk_body(k, carry):
        p, fg = carry
        pltpu.make_async_copy(
            entT_hbm.at[:, pl.ds(0, CW)], bbuf.at[0], semb).wait()

        @pl.when(k + 1 < nck)
        def _():
            start_chunk(k + 1)

        cnt = cnt_v[pl.ds(k, L)][0]
        return passes(bbuf, k & 1, (c0 + k) * CW, CW - 1, cnt, p, fg)

    p, fg = lax.fori_loop(0, nck, chunk_body,
                          (jnp.int32(0), jnp.int32(0)))

    def _finish(p3, fg3):
        @pl.when((p3 & (L - 1)) > 0)
        def _():
            flush_group(p3 >> 4, fg3)

        tot = fg3 + jnp.where((p3 & (L - 1)) > 0, 1, 0)

        @pl.when(tot >= 1)
        def _():
            pltpu.make_async_copy(stage.at[0], mat_hbm.at[0], semf).wait()

        @pl.when(tot >= 2)
        def _():
            pltpu.make_async_copy(stage.at[0], mat_hbm.at[0], semf).wait()

    @pl.when(wid == NW - 1)
    def _():
        pltpu.sync_copy(tailT_hbm, tbuf.at[0])
        cntt = cnt_v[pl.ds(nck, L)][0]
        p2, fg2 = passes(tbuf, 0, TAIL0, CH - 1, cntt, p, fg)
        _finish(p2, fg2)

    @pl.when(wid != NW - 1)
    def _():
        _finish(p, fg)


def _score_body(nch, hpos_hbm, tpos_hbm, rrow_hbm, rpar_hbm, mat_hbm,
                rel_hbm, out_hbm, hidx, tidx, ridx, rpar, hbuf, tbuf, rbuf,
                sbuf, sem):
    wid = lax.axis_index("s") * NC + lax.axis_index("c")
    base = wid * (nch * CH)

    pltpu.sync_copy(hpos_hbm.at[wid], hidx)
    pltpu.sync_copy(tpos_hbm.at[wid], tidx)
    pltpu.sync_copy(rrow_hbm.at[wid], ridx)
    pltpu.sync_copy(rpar_hbm.at[wid], rpar)

    iota = lax.iota(jnp.int32, L)

    def gather(c, slot):
        return (
            pltpu.async_copy(mat_hbm.at[hidx.at[c]], hbuf.at[slot], sem),
            pltpu.async_copy(mat_hbm.at[tidx.at[c]], tbuf.at[slot], sem),
            pltpu.async_copy(rel_hbm.at[ridx.at[c]], rbuf.at[slot], sem),
        )

    cps = gather(0, 0)
    for c in range(nch):
        slot = c % 2
        for cp in cps:
            cp.wait()
        if c + 1 < nch:
            cps = gather(c + 1, 1 - slot)

        for g in range(CH // L):
            row = iota + g * L
            pr = rpar[c, pl.ds(g * L, L)] << 6
            zeros = jnp.zeros((L,), jnp.float32)

            def step(i, acc):
                sh, sr, st, xhr, xht, xrt = acc
                for u in range(UNROLL):
                    jv = jnp.full((L,), i * UNROLL + u, jnp.int32)
                    gh = plsc.load_gather(hbuf.at[slot], [row, jv])
                    gt = plsc.load_gather(tbuf.at[slot], [row, jv])
                    gr = plsc.load_gather(rbuf.at[slot], [row, pr + jv])
                    sh = sh + gh * gh
                    sr = sr + gr * gr
                    st = st + gt * gt
                    xhr = xhr + gh * gr
                    xht = xht + gh * gt
                    xrt = xrt + gr * gt
                return (sh, sr, st, xhr, xht, xrt)

            sh, sr, st, xhr, xht, xrt = lax.fori_loop(
                0, D // UNROLL, step, (zeros,) * 6)

            cap = jnp.full((L,), 1e12, jnp.float32)
            ih = jnp.minimum(_rsqrt(sh), cap)
            ir = jnp.minimum(_rsqrt(sr), cap)
            it = jnp.minimum(_rsqrt(st), cap)
            s2 = (sh * ih * ih + sr * ir * ir + st * it * it
                  + jnp.float32(2.0) * (xhr * ih * ir - xht * ih * it
                                        - xrt * ir * it))
            s2 = jnp.maximum(s2, jnp.float32(0.0))
            sbuf[pl.ds(c * CH + g * L, L)] = s2 * _rsqrt(s2)

    pltpu.sync_copy(sbuf, out_hbm.at[pl.ds(base, nch * CH)])


def kernel(head, relation, tail, entity_embeddings, relation_embeddings):
    b = head.shape[0]
    nch = b // (NW * CH)

    ent_all = jnp.concatenate([head, tail])
    order = jnp.argsort(ent_all)
    inv_order = jnp.argsort(order)
    sorted_ent = ent_all[order]

    per = NFULL // NW
    extra = NFULL - per * NW
    w = jnp.arange(NW + 1, dtype=jnp.int32)
    cstart = w * per + jnp.minimum(w, extra)
    nck = cstart[1:] - cstart[:-1]

    ck_of = jnp.minimum(sorted_ent >> 9, NFULL).astype(jnp.int32)
    counts = jnp.zeros((NFULL + 1,), jnp.int32).at[ck_of].add(1)
    csum = jnp.concatenate([jnp.zeros((1,), jnp.int32),
                            jnp.cumsum(counts, dtype=jnp.int32)])
    s = csum[cstart]
    s = s.at[NW].set(2 * b)
    n = s[1:] - s[:-1]
    mg = (n + (L - 1)) // L + 1
    g0 = jnp.concatenate([jnp.zeros((1,), jnp.int32),
                          jnp.cumsum(mg, dtype=jnp.int32)])[:NW]
    salign = (s[:-1] // 8) * 8
    doff = s[:-1] - salign

    meta = jnp.stack([cstart[:-1], nck, s[:-1], salign, doff, g0] +
                     [jnp.zeros((NW,), jnp.int32)] * 10, axis=1)

    cidx = jnp.minimum(cstart[:NW, None] + jnp.arange(96)[None, :], NFULL)
    cvalid = jnp.arange(96)[None, :] <= (cstart[1:, None] - cstart[:NW, None])
    cvalid = cvalid & (cstart[:NW, None] + jnp.arange(96)[None, :] <= NFULL)
    cnts = jnp.where(cvalid, counts[cidx], 0).astype(jnp.int32)
    cnts = cnts * jnp.where((jnp.arange(NW)[:, None] == NW - 1)
                            | (jnp.arange(96)[None, :]
                               < (cstart[1:, None] - cstart[:NW, None])),
                            1, 0)

    wk = jnp.where(ck_of < (per + 1) * extra,
                   ck_of // (per + 1),
                   extra + (ck_of - (per + 1) * extra) // per)
    wk = jnp.minimum(wk, NW - 1).astype(jnp.int32)
    j = jnp.arange(2 * b, dtype=jnp.int32)
    pos_sorted = g0[wk] * L + (j - s[wk])
    matrow = pos_sorted[inv_order]
    hpos = matrow[:b].reshape(NW, nch, CH)
    tpos = matrow[b:].reshape(NW, nch, CH)

    sent_pad = jnp.concatenate(
        [sorted_ent.astype(jnp.int32), jnp.zeros((SEG + 8,), jnp.int32)])

    entT = entity_embeddings.T
    tailT = jnp.pad(entity_embeddings[TAIL0:], ((0, CH - (NE - TAIL0)),
                                                (0, 0))).T

    sweep = functools.partial(
        pl.kernel,
        out_type=jax.ShapeDtypeStruct((MAXG, L, CH), jnp.float32),
        mesh=plsc.VectorSubcoreMesh(**_MESH),
        compiler_params=_CP,
        scratch_types=[
            pltpu.VMEM((SEG,), jnp.int32),
            pltpu.VMEM((L,), jnp.int32),
            pltpu.VMEM((96,), jnp.int32),
            pltpu.VMEM((2, D, CW), jnp.float32),
            pltpu.VMEM((1, D, CH), jnp.float32),
            pltpu.VMEM((4, L, CH), jnp.float32),
            pltpu.SemaphoreType.DMA,
            pltpu.SemaphoreType.DMA,
        ],
    )(_sweep_body)
    mat = sweep(sent_pad, meta, cnts, entT, tailT)
    mat2 = mat.reshape(MAXG * L, CH)

    nr = relation_embeddings.shape[0]
    rel2 = relation_embeddings.reshape(nr // 2, 2 * D)
    rrow = (relation >> 1).reshape(NW, nch, CH)
    rpar = (relation & 1).reshape(NW, nch, CH)

    score = functools.partial(
        pl.kernel,
        out_type=jax.ShapeDtypeStruct((b,), jnp.float32),
        mesh=plsc.VectorSubcoreMesh(**_MESH),
        compiler_params=_CP,
        scratch_types=[
            pltpu.VMEM((nch, CH), jnp.int32),
            pltpu.VMEM((nch, CH), jnp.int32),
            pltpu.VMEM((nch, CH), jnp.int32),
            pltpu.VMEM((nch, CH), jnp.int32),
            pltpu.VMEM((2, CH, CH), jnp.float32),
            pltpu.VMEM((2, CH, CH), jnp.float32),
            pltpu.VMEM((2, CH, 2 * D), jnp.float32),
            pltpu.VMEM((nch * CH,), jnp.float32),
            pltpu.SemaphoreType.DMA,
        ],
    )(functools.partial(_score_body, nch))
    return score(hpos, tpos, rrow, rpar, mat2, rel2)


# speedup vs baseline: 2.1861x; 1.1329x over previous
import functools

import jax
import jax.numpy as jnp
from jax import lax
from jax.experimental import pallas as pl
from jax.experimental.pallas import tpu as pltpu
from jax.experimental.pallas import tpu_sc as plsc

D = 64
NC = 2
NS = 16
NW = NC * NS
CH = 128
L = 16
CW = 512
NE = 1000000
NFULL = NE // CW
TAIL0 = NFULL * CW
SEG = 2664
MAXG = 2112
UNROLL = 4

_CP = pltpu.CompilerParams(needs_layout_passes=False, use_tc_tiling_on_sc=True)
_MESH = dict(core_axis_name="c", subcore_axis_name="s")


def _rsqrt(x):
    i = plsc.bitcast(x, jnp.int32)
    i = jnp.int32(0x5F3759DF) - (i >> 1)
    y = plsc.bitcast(i, jnp.float32)
    hx = x * jnp.float32(0.5)
    for _ in range(3):
        y = y * (jnp.float32(1.5) - hx * y * y)
    return y


def _sweep_body(sent_hbm, meta_hbm, cnts_hbm, entT_hbm, tailT_hbm, mat_hbm,
                sent_v, meta_v, cnt_v, bbuf, tbuf, stage, semb, semf):
    wid = lax.axis_index("s") * NC + lax.axis_index("c")
    pltpu.sync_copy(meta_hbm.at[wid], meta_v)
    mv = meta_v[pl.ds(0, L)]
    c0 = mv[0]
    nck = mv[1]
    doff = mv[4]
    salign = mv[3]
    g0 = mv[5]

    pltpu.sync_copy(cnts_hbm.at[wid], cnt_v)
    pltpu.sync_copy(
        sent_hbm.at[pl.ds(pl.multiple_of(salign, 8), SEG)], sent_v)

    iota = lax.iota(jnp.int32, L)

    def flush_group(g, fg):
        @pl.when(fg >= 2)
        def _():
            pltpu.make_async_copy(stage.at[0], mat_hbm.at[0], semf).wait()
        pltpu.async_copy(stage.at[g & 3], mat_hbm.at[g0 + g], semf)

    def passes(buf, slot_s, start, wmax, cnt, p, fg):
        slotv = jnp.full((L,), slot_s, jnp.int32)

        def pass_body(ps, fg_):
            t0 = p + ps * L
            rem = jnp.minimum(cnt - ps * L, L)
            ev = sent_v[pl.ds(doff + t0, L)]
            cv = jnp.clip(ev - start, 0, wmax)
            tv = t0 + iota
            ssv = (tv >> 4) & 3
            rowv = tv & (L - 1)

            def jstep(i, _):
                for u in range(UNROLL):
                    jv = (jnp.full((L,), i * UNROLL + u, jnp.int32)
                          + iota) & (D - 1)
                    g = plsc.load_gather(buf, [slotv, jv, cv])
                    plsc.store_scatter(stage, [ssv, rowv, jv], g)
                return 0

            lax.fori_loop(0, D // UNROLL, jstep, 0)

            def doflush(fg2):
                flush_group(fg2, fg2)
                return fg2 + 1

            return lax.cond(((t0 + rem) >> 4) > fg_, doflush,
                            lambda fg2: fg2, fg_)

        fgo = lax.fori_loop(0, (cnt + L - 1) // L, pass_body, fg)
        return (p + cnt, fgo)

    def start_chunk(k):
        off = pl.multiple_of((c0 + k) * CW, CW)
        return pltpu.async_copy(
            entT_hbm.at[:, pl.ds(off, CW)], bbuf.at[k & 1], semb)

    @pl.when(nck > 0)
    def _():
        start_chunk(0)

    def chunk_body(k, carry):
        p, fg = carry
        pltpu.make_async_copy(
            entT_hbm.at[:, pl.ds(0, CW)], bbuf.at[0], semb).wait()

        @pl.when(k + 1 < nck)
        def _():
            start_chunk(k + 1)

        cnt = cnt_v[pl.ds(k, L)][0]
        return passes(bbuf, k & 1, (c0 + k) * CW, CW - 1, cnt, p, fg)

    p, fg = lax.fori_loop(0, nck, chunk_body,
                          (jnp.int32(0), jnp.int32(0)))

    def _finish(p3, fg3):
        @pl.when((p3 & (L - 1)) > 0)
        def _():
            flush_group(p3 >> 4, fg3)

        tot = fg3 + jnp.where((p3 & (L - 1)) > 0, 1, 0)

        @pl.when(tot >= 1)
        def _():
            pltpu.make_async_copy(stage.at[0], mat_hbm.at[0], semf).wait()

        @pl.when(tot >= 2)
        def _():
            pltpu.make_async_copy(stage.at[0], mat_hbm.at[0], semf).wait()

    @pl.when(wid == NW - 1)
    def _():
        pltpu.sync_copy(tailT_hbm, tbuf.at[0])
        cntt = cnt_v[pl.ds(nck, L)][0]
        p2, fg2 = passes(tbuf, 0, TAIL0, CH - 1, cntt, p, fg)
        _finish(p2, fg2)

    @pl.when(wid != NW - 1)
    def _():
        _finish(p, fg)


def _score_body(nch, hpos_hbm, tpos_hbm, rrow_hbm, rpar_hbm, mat_hbm,
                rel_hbm, out_hbm, hidx, tidx, ridx, rpar, hbuf, tbuf, rbuf,
                sbuf, sem):
    wid = lax.axis_index("s") * NC + lax.axis_index("c")
    base = wid * (nch * CH)

    pltpu.sync_copy(hpos_hbm.at[wid], hidx)
    pltpu.sync_copy(tpos_hbm.at[wid], tidx)
    pltpu.sync_copy(rrow_hbm.at[wid], ridx)
    pltpu.sync_copy(rpar_hbm.at[wid], rpar)

    iota = lax.iota(jnp.int32, L)

    def gather(c, slot):
        return (
            pltpu.async_copy(mat_hbm.at[hidx.at[c]], hbuf.at[slot], sem),
            pltpu.async_copy(mat_hbm.at[tidx.at[c]], tbuf.at[slot], sem),
            pltpu.async_copy(rel_hbm.at[ridx.at[c]], rbuf.at[slot], sem),
        )

    cps = gather(0, 0)
    for c in range(nch):
        slot = c % 2
        for cp in cps:
            cp.wait()
        if c + 1 < nch:
            cps = gather(c + 1, 1 - slot)

        for g in range(CH // L):
            row = iota + g * L
            pr = rpar[c, pl.ds(g * L, L)] << 6
            zeros = jnp.zeros((L,), jnp.float32)

            def step(i, acc):
                sh, sr, st, xhr, xht, xrt = acc
                for u in range(UNROLL):
                    jv = (jnp.full((L,), i * UNROLL + u, jnp.int32)
                          + iota) & (D - 1)
                    gh = plsc.load_gather(hbuf.at[slot], [row, jv])
                    gt = plsc.load_gather(tbuf.at[slot], [row, jv])
                    gr = plsc.load_gather(rbuf.at[slot], [row, pr + jv])
                    sh = sh + gh * gh
                    sr = sr + gr * gr
                    st = st + gt * gt
                    xhr = xhr + gh * gr
                    xht = xht + gh * gt
                    xrt = xrt + gr * gt
                return (sh, sr, st, xhr, xht, xrt)

            sh, sr, st, xhr, xht, xrt = lax.fori_loop(
                0, D // UNROLL, step, (zeros,) * 6)

            cap = jnp.full((L,), 1e12, jnp.float32)
            ih = jnp.minimum(_rsqrt(sh), cap)
            ir = jnp.minimum(_rsqrt(sr), cap)
            it = jnp.minimum(_rsqrt(st), cap)
            s2 = (sh * ih * ih + sr * ir * ir + st * it * it
                  + jnp.float32(2.0) * (xhr * ih * ir - xht * ih * it
                                        - xrt * ir * it))
            s2 = jnp.maximum(s2, jnp.float32(0.0))
            sbuf[pl.ds(c * CH + g * L, L)] = s2 * _rsqrt(s2)

    pltpu.sync_copy(sbuf, out_hbm.at[pl.ds(base, nch * CH)])


def kernel(head, relation, tail, entity_embeddings, relation_embeddings):
    b = head.shape[0]
    nch = b // (NW * CH)

    ent_all = jnp.concatenate([head, tail])
    order = jnp.argsort(ent_all)
    inv_order = jnp.argsort(order)
    sorted_ent = ent_all[order]

    per = NFULL // NW
    extra = NFULL - per * NW
    w = jnp.arange(NW + 1, dtype=jnp.int32)
    cstart = w * per + jnp.minimum(w, extra)
    nck = cstart[1:] - cstart[:-1]

    ck_of = jnp.minimum(sorted_ent >> 9, NFULL).astype(jnp.int32)
    counts = jnp.zeros((NFULL + 1,), jnp.int32).at[ck_of].add(1)
    csum = jnp.concatenate([jnp.zeros((1,), jnp.int32),
                            jnp.cumsum(counts, dtype=jnp.int32)])
    s = csum[cstart]
    s = s.at[NW].set(2 * b)
    n = s[1:] - s[:-1]
    mg = (n + (L - 1)) // L + 1
    g0 = jnp.concatenate([jnp.zeros((1,), jnp.int32),
                          jnp.cumsum(mg, dtype=jnp.int32)])[:NW]
    salign = (s[:-1] // 8) * 8
    doff = s[:-1] - salign

    meta = jnp.stack([cstart[:-1], nck, s[:-1], salign, doff, g0] +
                     [jnp.zeros((NW,), jnp.int32)] * 10, axis=1)

    cidx = jnp.minimum(cstart[:NW, None] + jnp.arange(96)[None, :], NFULL)
    cvalid = jnp.arange(96)[None, :] <= (cstart[1:, None] - cstart[:NW, None])
    cvalid = cvalid & (cstart[:NW, None] + jnp.arange(96)[None, :] <= NFULL)
    cnts = jnp.where(cvalid, counts[cidx], 0).astype(jnp.int32)
    cnts = cnts * jnp.where((jnp.arange(NW)[:, None] == NW - 1)
                            | (jnp.arange(96)[None, :]
                               < (cstart[1:, None] - cstart[:NW, None])),
                            1, 0)

    wk = jnp.where(ck_of < (per + 1) * extra,
                   ck_of // (per + 1),
                   extra + (ck_of - (per + 1) * extra) // per)
    wk = jnp.minimum(wk, NW - 1).astype(jnp.int32)
    j = jnp.arange(2 * b, dtype=jnp.int32)
    pos_sorted = g0[wk] * L + (j - s[wk])
    matrow = pos_sorted[inv_order]
    hpos = matrow[:b].reshape(NW, nch, CH)
    tpos = matrow[b:].reshape(NW, nch, CH)

    sent_pad = jnp.concatenate(
        [sorted_ent.astype(jnp.int32), jnp.zeros((SEG + 8,), jnp.int32)])

    entT = entity_embeddings.T
    tailT = jnp.pad(entity_embeddings[TAIL0:], ((0, CH - (NE - TAIL0)),
                                                (0, 0))).T

    sweep = functools.partial(
        pl.kernel,
        out_type=jax.ShapeDtypeStruct((MAXG, L, CH), jnp.float32),
        mesh=plsc.VectorSubcoreMesh(**_MESH),
        compiler_params=_CP,
        scratch_types=[
            pltpu.VMEM((SEG,), jnp.int32),
            pltpu.VMEM((L,), jnp.int32),
            pltpu.VMEM((96,), jnp.int32),
            pltpu.VMEM((2, D, CW), jnp.float32),
            pltpu.VMEM((1, D, CH), jnp.float32),
            pltpu.VMEM((4, L, CH), jnp.float32),
            pltpu.SemaphoreType.DMA,
            pltpu.SemaphoreType.DMA,
        ],
    )(_sweep_body)
    mat = sweep(sent_pad, meta, cnts, entT, tailT)
    mat2 = mat.reshape(MAXG * L, CH)

    nr = relation_embeddings.shape[0]
    rel2 = relation_embeddings.reshape(nr // 2, 2 * D)
    rrow = (relation >> 1).reshape(NW, nch, CH)
    rpar = (relation & 1).reshape(NW, nch, CH)

    score = functools.partial(
        pl.kernel,
        out_type=jax.ShapeDtypeStruct((b,), jnp.float32),
        mesh=plsc.VectorSubcoreMesh(**_MESH),
        compiler_params=_CP,
        scratch_types=[
            pltpu.VMEM((nch, CH), jnp.int32),
            pltpu.VMEM((nch, CH), jnp.int32),
            pltpu.VMEM((nch, CH), jnp.int32),
            pltpu.VMEM((nch, CH), jnp.int32),
            pltpu.VMEM((2, CH, CH), jnp.float32),
            pltpu.VMEM((2, CH, CH), jnp.float32),
            pltpu.VMEM((2, CH, 2 * D), jnp.float32),
            pltpu.VMEM((nch * CH,), jnp.float32),
            pltpu.SemaphoreType.DMA,
        ],
    )(functools.partial(_score_body, nch))
    return score(hpos, tpos, rrow, rpar, mat2, rel2)


# speedup vs baseline: 4.2339x; 1.9368x over previous
import functools

import jax
import jax.numpy as jnp
from jax import lax
from jax.experimental import pallas as pl
from jax.experimental.pallas import tpu as pltpu
from jax.experimental.pallas import tpu_sc as plsc

D = 64
NC = 2
NS = 16
NW = NC * NS
CH = 128
L = 16
CW = 512
NE = 1000000
NFULL = NE // CW
TAIL0 = NFULL * CW
BPW = 1024
GPW = BPW // L
MAXG = NW * GPW
SEG = BPW + L
UNROLL = 4

_CP = pltpu.CompilerParams(needs_layout_passes=False, use_tc_tiling_on_sc=True)
_MESH = dict(core_axis_name="c", subcore_axis_name="s")


def _rsqrt(x):
    i = plsc.bitcast(x, jnp.int32)
    i = jnp.int32(0x5F3759DF) - (i >> 1)
    y = plsc.bitcast(i, jnp.float32)
    hx = x * jnp.float32(0.5)
    for _ in range(3):
        y = y * (jnp.float32(1.5) - hx * y * y)
    return y


def _sweep_body(sent_hbm, entT_hbm, tailT_hbm, mat_hbm,
                sent_v, bbuf, tbuf, stage, semb, semf):
    wid = lax.axis_index("s") * NC + lax.axis_index("c")
    g0 = wid * GPW

    pltpu.sync_copy(
        sent_hbm.at[pl.ds(pl.multiple_of(wid * BPW, 8), SEG)], sent_v)

    iota = lax.iota(jnp.int32, L)

    e_first = sent_v[pl.ds(0, L)][0]
    e_last = sent_v[pl.ds(BPW - L, L)][L - 1]
    c_lo = e_first >> 9
    c_hi_all = e_last >> 9
    c_hi = jnp.minimum(c_hi_all, NFULL - 1)
    nck = jnp.maximum(c_hi - c_lo + 1, 0)

    def flush_group(g, fg):
        @pl.when(fg >= 2)
        def _():
            pltpu.make_async_copy(stage.at[0], mat_hbm.at[0], semf).wait()
        pltpu.async_copy(stage.at[g & 3], mat_hbm.at[g0 + g], semf)

    def _drain(fg3):
        @pl.when(fg3 >= 1)
        def _():
            pltpu.make_async_copy(stage.at[0], mat_hbm.at[0], semf).wait()

        @pl.when(fg3 >= 2)
        def _():
            pltpu.make_async_copy(stage.at[0], mat_hbm.at[0], semf).wait()

    def extract_chunk(cid, buf, slot_s, start, wmax, p, fg):
        slotv = jnp.full((L,), slot_s, jnp.int32)

        def cond(state):
            return state[2]

        def body(state):
            p_, fg_, _ = state
            ev = sent_v[pl.ds(p_, L)]
            cnt = plsc.all_reduce_population_count((ev >> 9) == cid)[0]
            cv = jnp.clip(ev - start, 0, wmax)
            tv = p_ + iota
            ssv = (tv >> 4) & 3
            rowv = tv & (L - 1)

            @pl.when(cnt > 0)
            def _():
                def jstep(i, _):
                    for u in range(UNROLL):
                        jv = (jnp.full((L,), i * UNROLL + u, jnp.int32)
                              + iota) & (D - 1)
                        g = plsc.load_gather(buf, [slotv, jv, cv])
                        plsc.store_scatter(stage, [ssv, rowv, jv], g)
                    return 0

                lax.fori_loop(0, D // UNROLL, jstep, 0)

            newp = p_ + cnt

            def doflush(fg2):
                flush_group(fg2, fg2)
                return fg2 + 1

            fg_ = lax.cond((newp >> 4) > fg_, doflush, lambda f: f, fg_)
            return (newp, fg_, (cnt == L) & (newp < BPW))

        p, fg, _ = lax.while_loop(cond, body, (p, fg, True))
        return p, fg

    def start_chunk(k, slot):
        off = pl.multiple_of((c_lo + k) * CW, CW)
        return pltpu.async_copy(
            entT_hbm.at[:, pl.ds(off, CW)], bbuf.at[slot], semb)

    @pl.when(nck > 0)
    def _():
        start_chunk(0, 0)

    @pl.when(nck > 1)
    def _():
        start_chunk(1, 1)

    def chunk_body(k, carry):
        p, fg = carry
        pltpu.make_async_copy(
            entT_hbm.at[:, pl.ds(0, CW)], bbuf.at[0], semb).wait()

        @pl.when(k + 2 < nck)
        def _():
            start_chunk(k + 2, (k + 2) % 3)

        return extract_chunk(c_lo + k, bbuf, k % 3, (c_lo + k) * CW,
                             CW - 1, p, fg)

    p, fg = lax.fori_loop(0, nck, chunk_body,
                          (jnp.int32(0), jnp.int32(0)))

    @pl.when(c_hi_all >= NFULL)
    def _():
        pltpu.sync_copy(tailT_hbm, tbuf.at[0])
        p2, fg2 = extract_chunk(jnp.int32(NFULL), tbuf, 0, TAIL0,
                                CH - 1, p, fg)
        del p2
        _drain(fg2)

    @pl.when(c_hi_all < NFULL)
    def _():
        _drain(fg)


def _score_body(nch, hpos_hbm, tpos_hbm, rrow_hbm, rpar_hbm, mat_hbm,
                rel_hbm, out_hbm, hidx, tidx, ridx, rpar, hbuf, tbuf, rbuf,
                sbuf, sem):
    wid = lax.axis_index("s") * NC + lax.axis_index("c")
    base = wid * (nch * CH)

    pltpu.sync_copy(hpos_hbm.at[wid], hidx)
    pltpu.sync_copy(tpos_hbm.at[wid], tidx)
    pltpu.sync_copy(rrow_hbm.at[wid], ridx)
    pltpu.sync_copy(rpar_hbm.at[wid], rpar)

    iota = lax.iota(jnp.int32, L)

    def gather(c, slot):
        return (
            pltpu.async_copy(mat_hbm.at[hidx.at[c]], hbuf.at[slot], sem),
            pltpu.async_copy(mat_hbm.at[tidx.at[c]], tbuf.at[slot], sem),
            pltpu.async_copy(rel_hbm.at[ridx.at[c]], rbuf.at[slot], sem),
        )

    cps = gather(0, 0)
    for c in range(nch):
        slot = c % 2
        for cp in cps:
            cp.wait()
        if c + 1 < nch:
            cps = gather(c + 1, 1 - slot)

        for g in range(CH // L):
            row = iota + g * L
            pr = rpar[c, pl.ds(g * L, L)] << 6
            zeros = jnp.zeros((L,), jnp.float32)

            def step(i, acc):
                sh, sr, st, xhr, xht, xrt = acc
                for u in range(UNROLL):
                    jv = (jnp.full((L,), i * UNROLL + u, jnp.int32)
                          + iota) & (D - 1)
                    gh = plsc.load_gather(hbuf.at[slot], [row, jv])
                    gt = plsc.load_gather(tbuf.at[slot], [row, jv])
                    gr = plsc.load_gather(rbuf.at[slot], [row, pr + jv])
                    sh = sh + gh * gh
                    sr = sr + gr * gr
                    st = st + gt * gt
                    xhr = xhr + gh * gr
                    xht = xht + gh * gt
                    xrt = xrt + gr * gt
                return (sh, sr, st, xhr, xht, xrt)

            sh, sr, st, xhr, xht, xrt = lax.fori_loop(
                0, D // UNROLL, step, (zeros,) * 6)

            cap = jnp.full((L,), 1e12, jnp.float32)
            ih = jnp.minimum(_rsqrt(sh), cap)
            ir = jnp.minimum(_rsqrt(sr), cap)
            it = jnp.minimum(_rsqrt(st), cap)
            s2 = (sh * ih * ih + sr * ir * ir + st * it * it
                  + jnp.float32(2.0) * (xhr * ih * ir - xht * ih * it
                                        - xrt * ir * it))
            s2 = jnp.maximum(s2, jnp.float32(0.0))
            sbuf[pl.ds(c * CH + g * L, L)] = s2 * _rsqrt(s2)

    pltpu.sync_copy(sbuf, out_hbm.at[pl.ds(base, nch * CH)])


def kernel(head, relation, tail, entity_embeddings, relation_embeddings):
    b = head.shape[0]
    nch = b // (NW * CH)

    ent_all = jnp.concatenate([head, tail])
    order = jnp.argsort(ent_all)
    inv_order = jnp.argsort(order).astype(jnp.int32)
    sorted_ent = ent_all[order]
    hpos = inv_order[:b].reshape(NW, nch, CH)
    tpos = inv_order[b:].reshape(NW, nch, CH)

    sent_pad = jnp.concatenate(
        [sorted_ent.astype(jnp.int32), jnp.zeros((SEG + 8,), jnp.int32)])

    entT = entity_embeddings.T
    tailT = jnp.pad(entity_embeddings[TAIL0:], ((0, CH - (NE - TAIL0)),
                                                (0, 0))).T

    sweep = functools.partial(
        pl.kernel,
        out_type=jax.ShapeDtypeStruct((MAXG, L, CH), jnp.float32),
        mesh=plsc.VectorSubcoreMesh(**_MESH),
        compiler_params=_CP,
        scratch_types=[
            pltpu.VMEM((SEG,), jnp.int32),
            pltpu.VMEM((3, D, CW), jnp.float32),
            pltpu.VMEM((1, D, CH), jnp.float32),
            pltpu.VMEM((4, L, CH), jnp.float32),
            pltpu.SemaphoreType.DMA,
            pltpu.SemaphoreType.DMA,
        ],
    )(_sweep_body)
    mat = sweep(sent_pad, entT, tailT)
    mat2 = mat.reshape(MAXG * L, CH)

    nr = relation_embeddings.shape[0]
    rel2 = relation_embeddings.reshape(nr // 2, 2 * D)
    rrow = (relation >> 1).reshape(NW, nch, CH)
    rpar = (relation & 1).reshape(NW, nch, CH)

    score = functools.partial(
        pl.kernel,
        out_type=jax.ShapeDtypeStruct((b,), jnp.float32),
        mesh=plsc.VectorSubcoreMesh(**_MESH),
        compiler_params=_CP,
        scratch_types=[
            pltpu.VMEM((nch, CH), jnp.int32),
            pltpu.VMEM((nch, CH), jnp.int32),
            pltpu.VMEM((nch, CH), jnp.int32),
            pltpu.VMEM((nch, CH), jnp.int32),
            pltpu.VMEM((2, CH, CH), jnp.float32),
            pltpu.VMEM((2, CH, CH), jnp.float32),
            pltpu.VMEM((2, CH, 2 * D), jnp.float32),
            pltpu.VMEM((nch * CH,), jnp.float32),
            pltpu.SemaphoreType.DMA,
        ],
    )(functools.partial(_score_body, nch))
    return score(hpos, tpos, rrow, rpar, mat2, rel2)
